# baseline (speedup 1.0000x reference)
import functools

import jax
import jax.numpy as jnp
from jax import lax
from jax.experimental import pallas as pl
from jax.experimental.pallas import tpu as pltpu
from jax.experimental.pallas import tpu_sc as plsc

VOCAB = 1000000
EMBED = 32
B = 16384
L = 50
NC = 2
NS = 16
NW = NC * NS
RW = B // NW
NCH = RW
NBUF = 4

_mesh = plsc.VectorSubcoreMesh(
    core_axis_name="c", subcore_axis_name="s", num_cores=NC, num_subcores=NS
)


@functools.partial(
    pl.kernel,
    out_type=jax.ShapeDtypeStruct((B * EMBED,), jnp.float32),
    mesh=_mesh,
    scratch_types=[
        pltpu.VMEM((NCH, L), jnp.int32),
        [pltpu.VMEM((L, EMBED), jnp.bfloat16) for _ in range(NBUF)],
        pltpu.VMEM((RW * EMBED,), jnp.float32),
        [pltpu.SemaphoreType.DMA for _ in range(NBUF)],
    ],
    compiler_params=pltpu.CompilerParams(
        use_tc_tiling_on_sc=False, needs_layout_passes=False),
)
def _sc_embed_mean(table_hbm, idx_hbm, out_hbm, idx_v, gs, out_v, sems):
    wid = lax.axis_index("c") * NS + lax.axis_index("s")
    pltpu.sync_copy(idx_hbm.at[pl.ds(wid * RW, RW)], idx_v)

    def start(c, b):
        pltpu.async_copy(table_hbm.at[idx_v.at[c]], gs[b], sems[b])

    def wait(b):
        pltpu.make_async_copy(table_hbm.at[idx_v.at[0]], gs[b], sems[b]).wait()

    scale = jnp.float32(1.0 / L)
    himask = jnp.int32(-65536)

    def row(g, j):
        v = plsc.bitcast(g[j, pl.ds(0, EMBED)], jnp.int32)
        even = plsc.bitcast(lax.shift_left(v, 16), jnp.float32)
        odd = plsc.bitcast(lax.bitwise_and(v, himask), jnp.float32)
        return even, odd

    def process(c, b):
        g = gs[b]
        acc0, acc1 = row(g, 0)
        for j in range(1, L):
            e, o = row(g, j)
            acc0 = acc0 + e
            acc1 = acc1 + o
        out_v[pl.ds(c * EMBED, 16)] = acc0 * scale
        out_v[pl.ds(c * EMBED + 16, 16)] = acc1 * scale

    for b in range(NBUF):
        start(b, b)

    @pl.loop(0, NCH - NBUF, step=NBUF)
    def _(c):
        for b in range(NBUF):
            wait(b)
            process(c + b, b)
            start(c + b + NBUF, b)

    for b in range(NBUF):
        wait(b)
        process(NCH - NBUF + b, b)

    pltpu.sync_copy(out_v, out_hbm.at[pl.ds(wid * (RW * EMBED), RW * EMBED)])


def kernel(item_tensors, table):
    table_bf = table.astype(jnp.bfloat16)
    out = _sc_embed_mean(table_bf, item_tensors)
    out = out.reshape(B, 2, EMBED // 2).transpose(0, 2, 1).reshape(B, EMBED)
    return out


# speedup vs baseline: 1.3203x; 1.3203x over previous
import functools

import jax
import jax.numpy as jnp
from jax import lax
from jax.experimental import pallas as pl
from jax.experimental.pallas import tpu as pltpu
from jax.experimental.pallas import tpu_sc as plsc

VOCAB = 1000000
EMBED = 32
B = 16384
L = 50
NC = 2
NS = 16
NW = NC * NS
RW = B // NW
NCH = RW
NBUF = 4

_mesh = plsc.VectorSubcoreMesh(
    core_axis_name="c", subcore_axis_name="s", num_cores=NC, num_subcores=NS
)


@functools.partial(
    pl.kernel,
    out_type=jax.ShapeDtypeStruct((B * EMBED,), jnp.float32),
    mesh=_mesh,
    scratch_types=[
        pltpu.VMEM((NCH // 2, 2 * L), jnp.int32),
        [pltpu.VMEM((2 * L, EMBED), jnp.float32) for _ in range(NBUF)],
        pltpu.VMEM((RW * EMBED,), jnp.float32),
        [pltpu.SemaphoreType.DMA for _ in range(NBUF)],
    ],
    compiler_params=pltpu.CompilerParams(use_tc_tiling_on_sc=False),
)
def _sc_embed_mean(table_hbm, idx_hbm, out_hbm, idx_v, gs, out_v, sems):
    wid = lax.axis_index("c") * NS + lax.axis_index("s")
    pltpu.sync_copy(idx_hbm.at[pl.ds(wid * (RW // 2), RW // 2)], idx_v)

    def start(c, b):
        pltpu.async_copy(table_hbm.at[idx_v.at[c]], gs[b], sems[b])

    def wait(b):
        pltpu.make_async_copy(table_hbm.at[idx_v.at[0]], gs[b], sems[b]).wait()

    scale = jnp.float32(1.0 / L)

    def process(c, b):
        g = gs[b]
        for r in range(2):
            b0 = r * L
            acc0 = g[b0, pl.ds(0, 16)]
            acc1 = g[b0, pl.ds(16, 16)]
            for j in range(1, L):
                acc0 = acc0 + g[b0 + j, pl.ds(0, 16)]
                acc1 = acc1 + g[b0 + j, pl.ds(16, 16)]
            out_v[pl.ds((2 * c + r) * EMBED, 16)] = acc0 * scale
            out_v[pl.ds((2 * c + r) * EMBED + 16, 16)] = acc1 * scale

    for b in range(NBUF):
        start(b, b)

    @pl.loop(0, NCH // 2 - NBUF, step=NBUF)
    def _(c):
        for b in range(NBUF):
            wait(b)
            process(c + b, b)
            start(c + b + NBUF, b)

    for b in range(NBUF):
        wait(b)
        process(NCH // 2 - NBUF + b, b)

    pltpu.sync_copy(out_v, out_hbm.at[pl.ds(wid * (RW * EMBED), RW * EMBED)])


def kernel(item_tensors, table):
    out = _sc_embed_mean(table, item_tensors.reshape(B // 2, 2 * L))
    return out.reshape(B, EMBED)
